# baseline (speedup 1.0000x reference)
import functools

import jax
import jax.numpy as jnp
from jax import lax
from jax.experimental import pallas as pl
from jax.experimental.pallas import tpu as pltpu
from jax.experimental.pallas import tpu_sc as plsc

K = 100
C = 1000
B = 1024
NC = 2
NS = 16
NW = NC * NS
RPW = B // NW
LANES = 16
NFULL = C // LANES
TAIL = C - NFULL * LANES


CPAD = 1024


def _softmax_body(p_ref, w_ref):
    p = p_ref[...]
    m = jnp.max(p, axis=1, keepdims=True)
    e = jnp.exp(p - m)
    w = e / jnp.sum(e, axis=1, keepdims=True)
    w_ref[...] = jnp.concatenate(
        [w, jnp.zeros((K, CPAD - C), jnp.float32)], axis=1)


def _softmax_rows(params):
    return pl.pallas_call(
        _softmax_body,
        out_shape=jax.ShapeDtypeStruct((K, CPAD), jnp.float32),
    )(params)


_sc_mesh = plsc.VectorSubcoreMesh(core_axis_name="c", subcore_axis_name="s")


@functools.partial(
    pl.kernel,
    mesh=_sc_mesh,
    out_type=jax.ShapeDtypeStruct((NW, LANES), jnp.float32),
    scratch_types=[
        pltpu.VMEM((RPW,), jnp.int32),
        pltpu.VMEM((RPW, CPAD), jnp.float32),
        pltpu.VMEM((RPW * C,), jnp.float32),
        pltpu.VMEM((LANES,), jnp.float32),
        pltpu.SemaphoreType.DMA,
    ],
)
def _sc_weighted_sum(w_hbm, losses_hbm, idx_hbm, out_hbm,
                     idx_v, rows_v, l_v, out_v, sem):
    wid = lax.axis_index("c") * NS + lax.axis_index("s")
    base = wid * RPW
    pltpu.sync_copy(idx_hbm.at[pl.ds(base, RPW)], idx_v)
    pltpu.async_copy(w_hbm.at[idx_v], rows_v, sem).wait()
    pltpu.sync_copy(losses_hbm.at[pl.ds(base * C, RPW * C)], l_v)

    lane = lax.broadcasted_iota(jnp.int32, (LANES,), 0)

    def row_body(r, acc):
        off = r * C
        for j in range(NFULL):
            acc = acc + (rows_v[r, pl.ds(j * LANES, LANES)]
                         * l_v[pl.ds(off + j * LANES, LANES)])
        t = (rows_v[r, pl.ds(C - LANES, LANES)]
             * l_v[pl.ds(off + C - LANES, LANES)])
        return acc + jnp.where(lane >= LANES - TAIL, t, 0.0)

    acc = lax.fori_loop(0, RPW, row_body, jnp.zeros((LANES,), jnp.float32))
    out_v[...] = acc
    pltpu.sync_copy(out_v, out_hbm.at[wid])


def kernel(losses, inputs_idx, params):
    w = _softmax_rows(params)
    parts = _sc_weighted_sum(w, losses, inputs_idx.astype(jnp.int32))
    return jnp.sum(parts)
